# baseline (speedup 1.0000x reference)
import functools
import jax
import jax.numpy as jnp
from jax import lax
from jax.experimental import pallas as pl
from jax.experimental.pallas import tpu as pltpu
from jax.experimental.pallas import tpu_sc as plsc

N_ROWS = 100000
D = 128
NSEG = 512
NLANE = 16
NVREG = D // NLANE
NC = 2
NS = 16
NW = NC * NS
SEG_PER_W = NSEG // NW
STARTS_PER_W = 32
CHUNK = 232
CHUNKB = CHUNK + 8
NBUF = 4
NSEARCH = 17
SLOT = 6400
NSLOT = 16
LAST_SLOT = N_ROWS - SLOT * (NSLOT - 1)

NEG_INF = float("-inf")


def _worker_id():
    return lax.axis_index("s") * NC + lax.axis_index("c")


def _sc_starts(idx_hbm, idx_v, pos_v, probes_v, out_v, sem):
    wid = _worker_id()
    lane = lax.broadcasted_iota(jnp.int32, (NLANE,), 0)

    pos_v[pl.ds(0, NLANE)] = jnp.minimum((lane + 1) * SLOT,
                                         SLOT * (NSLOT - 1))
    copy = pltpu.make_async_copy(idx_hbm.at[pos_v], probes_v, sem)
    copy.start()
    copy.wait()
    big = jnp.full((NLANE,), 1 << 30, jnp.int32)
    vals = jnp.where(lane < NSLOT - 1, probes_v[pl.ds(0, NLANE)], big)

    q_min = wid * SEG_PER_W
    q_max = q_min + SEG_PER_W
    k_lo = plsc.all_reduce_population_count(vals < q_min)[0]
    k_hi = plsc.all_reduce_population_count(vals < q_max)[0]
    row_a = k_lo * SLOT
    row_b = jnp.minimum((k_hi + 1) * SLOT, N_ROWS)

    def copy_slot(k2, _):
        dst = pl.multiple_of((k2 - k_lo) * SLOT, 8)
        src = pl.multiple_of(k2 * SLOT, 8)

        @pl.when(k2 < NSLOT - 1)
        def _():
            pltpu.sync_copy(idx_hbm.at[pl.ds(src, SLOT)],
                            idx_v.at[pl.ds(dst, SLOT)])

        @pl.when(k2 >= NSLOT - 1)
        def _():
            pltpu.sync_copy(idx_hbm.at[pl.ds(src, LAST_SLOT)],
                            idx_v.at[pl.ds(dst, LAST_SLOT)])

        return 0

    lax.fori_loop(k_lo, k_hi + 1, copy_slot, 0)

    for half in range(2):
        q = q_min + half * NLANE + lane

        def step(_, carry):
            lo, hi = carry
            mid = jnp.maximum((lo + hi) // 2, row_a)
            probe = plsc.load_gather(idx_v, [mid - row_a])
            lt = probe < q
            return (jnp.where(lt, mid, lo), jnp.where(lt, hi, mid))

        lo0 = jnp.full((NLANE,), -1, jnp.int32) + row_a
        hi0 = jnp.full((NLANE,), 0, jnp.int32) + row_b
        _, hi = lax.fori_loop(0, NSEARCH, step, (lo0, hi0))
        out_v[pl.ds(half * NLANE, NLANE)] = hi


def _sc_segment_max(embs_hbm, out_hbm, starts_v, rows_v, out_v, sems):
    wid = _worker_id()
    seg_base = wid * SEG_PER_W

    neg = jnp.full((NLANE,), NEG_INF, jnp.float32)
    for j in range(SEG_PER_W):
        for u in range(NVREG):
            out_v[j, pl.ds(u * NLANE, NLANE)] = neg

    head = starts_v[pl.ds(0, NLANE)]
    tail = starts_v[pl.ds(NLANE, NLANE)]
    r_lo = head[0]
    r_hi = tail[0]
    nchunks = (r_hi - r_lo + CHUNK - 1) // CHUNK

    def chunk_base(k):
        b0 = r_lo + k * CHUNK
        base_c = jnp.minimum((b0 // 8) * 8, N_ROWS - CHUNKB)
        return b0, pl.multiple_of(base_c, 8)

    def start_chunk(k):
        _, base_c = chunk_base(k)
        pltpu.make_async_copy(embs_hbm.at[pl.ds(base_c, CHUNKB)],
                              rows_v.at[k % NBUF], sems.at[k % NBUF]).start()

    for p in range(NBUF - 1):
        @pl.when(p < nchunks)
        def _():
            start_chunk(p)

    def per_chunk(m, carry):
        j = carry[0]
        acc0 = carry[1:]
        par = m % NBUF
        b0, base_c = chunk_base(m)
        b1 = jnp.minimum(b0 + CHUNK, r_hi)

        pltpu.make_async_copy(embs_hbm.at[pl.ds(base_c, CHUNKB)],
                              rows_v.at[par], sems.at[par]).wait()

        @pl.when(m + NBUF - 1 < nchunks)
        def _():
            start_chunk(m + NBUF - 1)

        n_started = plsc.all_reduce_population_count(head < b1)
        jB = n_started[0] - 1

        def per_segment(j2, acc):
            bounds = starts_v[pl.ds(j2, NLANE)]
            g0 = jnp.maximum(bounds[0], b0) - base_c
            g1 = jnp.minimum(bounds[1], b1) - base_c

            def row_body(r, acc):
                return tuple(
                    jnp.maximum(acc[u],
                                rows_v[par, r, pl.ds(u * NLANE, NLANE)])
                    for u in range(NVREG))

            UNROLL = 4
            ngrp = (g1 - g0) // UNROLL

            def grp_body(k, acc):
                r = g0 + k * UNROLL
                for i in range(UNROLL):
                    acc = row_body(r + i, acc)
                return acc

            acc = lax.fori_loop(0, ngrp, grp_body, acc)
            acc = lax.fori_loop(g0 + ngrp * UNROLL, g1, row_body, acc)
            for u in range(NVREG):
                out_v[j2, pl.ds(u * NLANE, NLANE)] = acc[u]
            done = bounds[1] <= b1
            return tuple(jnp.where(done, neg, a) for a in acc)

        acc1 = lax.fori_loop(j, jB + 1, per_segment, acc0)
        endB = starts_v[pl.ds(jB, NLANE)]
        j_next = jB + (endB[1] <= b1).astype(jnp.int32)
        return (j_next,) + acc1

    init = (jnp.int32(0),) + tuple(neg for _ in range(NVREG))
    lax.fori_loop(0, nchunks, per_chunk, init)

    pltpu.sync_copy(out_v, out_hbm.at[pl.ds(seg_base, SEG_PER_W)])


def _sc_fused(embs_hbm, idx_hbm, out_hbm, starts_v, out_v, pos_v, probes_v,
              sem, sems):
    def phase_a(idx_v):
        _sc_starts(idx_hbm, idx_v, pos_v, probes_v, starts_v, sem)

    pl.run_scoped(phase_a, pltpu.VMEM((NSLOT * SLOT,), jnp.int32))

    def phase_b(rows_v):
        _sc_segment_max(embs_hbm, out_hbm, starts_v, rows_v, out_v, sems)

    pl.run_scoped(phase_b, pltpu.VMEM((NBUF, CHUNKB, D), jnp.float32))


@jax.jit
def kernel(embs, batch_idx):
    idx = batch_idx.astype(jnp.int32)
    mesh = plsc.VectorSubcoreMesh(core_axis_name="c", subcore_axis_name="s")

    run = pl.kernel(
        _sc_fused,
        mesh=mesh,
        out_type=jax.ShapeDtypeStruct((NSEG, D), jnp.float32),
        scratch_types=[
            pltpu.VMEM((STARTS_PER_W,), jnp.int32),
            pltpu.VMEM((SEG_PER_W, D), jnp.float32),
            pltpu.VMEM((NLANE,), jnp.int32),
            pltpu.VMEM((NLANE,), jnp.int32),
            pltpu.SemaphoreType.DMA,
            pltpu.SemaphoreType.DMA((NBUF,)),
        ],
        compiler_params=pltpu.CompilerParams(needs_layout_passes=False),
    )
    return run(embs, idx)


# speedup vs baseline: 1.0088x; 1.0088x over previous
import functools
import jax
import jax.numpy as jnp
from jax import lax
from jax.experimental import pallas as pl
from jax.experimental.pallas import tpu as pltpu
from jax.experimental.pallas import tpu_sc as plsc

N_ROWS = 100000
D = 128
NSEG = 512
NLANE = 16
NVREG = D // NLANE
NC = 2
NS = 16
NW = NC * NS
SEG_PER_W = NSEG // NW
STARTS_PER_W = 32
CHUNK = 320
CHUNKB = CHUNK + 8
NBUF = 3
HALF1 = 160
HALF2 = CHUNKB - HALF1
NSEARCH = 17
SLOT = 6400
NSLOT = 16
LAST_SLOT = N_ROWS - SLOT * (NSLOT - 1)

NEG_INF = float("-inf")


def _worker_id():
    return lax.axis_index("s") * NC + lax.axis_index("c")


def _sc_starts(idx_hbm, idx_v, pos_v, probes_v, out_v, sem):
    wid = _worker_id()
    lane = lax.broadcasted_iota(jnp.int32, (NLANE,), 0)

    pos_v[pl.ds(0, NLANE)] = jnp.minimum((lane + 1) * SLOT,
                                         SLOT * (NSLOT - 1))
    copy = pltpu.make_async_copy(idx_hbm.at[pos_v], probes_v, sem)
    copy.start()
    copy.wait()
    big = jnp.full((NLANE,), 1 << 30, jnp.int32)
    vals = jnp.where(lane < NSLOT - 1, probes_v[pl.ds(0, NLANE)], big)

    q_min = wid * SEG_PER_W
    q_max = q_min + SEG_PER_W
    k_lo = plsc.all_reduce_population_count(vals < q_min)[0]
    k_hi = plsc.all_reduce_population_count(vals < q_max)[0]
    row_a = k_lo * SLOT
    row_b = jnp.minimum((k_hi + 1) * SLOT, N_ROWS)

    def copy_slot(k2, _):
        dst = pl.multiple_of((k2 - k_lo) * SLOT, 8)
        src = pl.multiple_of(k2 * SLOT, 8)

        @pl.when(k2 < NSLOT - 1)
        def _():
            pltpu.sync_copy(idx_hbm.at[pl.ds(src, SLOT)],
                            idx_v.at[pl.ds(dst, SLOT)])

        @pl.when(k2 >= NSLOT - 1)
        def _():
            pltpu.sync_copy(idx_hbm.at[pl.ds(src, LAST_SLOT)],
                            idx_v.at[pl.ds(dst, LAST_SLOT)])

        return 0

    lax.fori_loop(k_lo, k_hi + 1, copy_slot, 0)

    for half in range(2):
        q = q_min + half * NLANE + lane

        def step(_, carry):
            lo, hi = carry
            mid = jnp.maximum((lo + hi) // 2, row_a)
            probe = plsc.load_gather(idx_v, [mid - row_a])
            lt = probe < q
            return (jnp.where(lt, mid, lo), jnp.where(lt, hi, mid))

        lo0 = jnp.full((NLANE,), -1, jnp.int32) + row_a
        hi0 = jnp.full((NLANE,), 0, jnp.int32) + row_b
        _, hi = lax.fori_loop(0, NSEARCH, step, (lo0, hi0))
        out_v[pl.ds(half * NLANE, NLANE)] = hi


def _sc_segment_max(embs_hbm, out_hbm, starts_v, rows_v, out_v, sems):
    wid = _worker_id()
    seg_base = wid * SEG_PER_W

    neg = jnp.full((NLANE,), NEG_INF, jnp.float32)
    for j in range(SEG_PER_W):
        for u in range(NVREG):
            out_v[j, pl.ds(u * NLANE, NLANE)] = neg

    head = starts_v[pl.ds(0, NLANE)]
    tail = starts_v[pl.ds(NLANE, NLANE)]
    r_lo = head[0]
    r_hi = tail[0]
    nchunks = (r_hi - r_lo + CHUNK - 1) // CHUNK

    def chunk_base(k):
        b0 = r_lo + k * CHUNK
        base_c = jnp.minimum((b0 // 8) * 8, N_ROWS - CHUNKB)
        return b0, pl.multiple_of(base_c, 8)

    def chunk_copies(k):
        _, base_c = chunk_base(k)
        buf = k % NBUF
        c1 = pltpu.make_async_copy(
            embs_hbm.at[pl.ds(base_c, HALF1)],
            rows_v.at[buf].at[pl.ds(0, HALF1)], sems.at[buf])
        c2 = pltpu.make_async_copy(
            embs_hbm.at[pl.ds(base_c + HALF1, HALF2)],
            rows_v.at[buf].at[pl.ds(HALF1, HALF2)], sems.at[buf])
        return c1, c2

    def start_chunk(k):
        c1, c2 = chunk_copies(k)
        c1.start()
        c2.start()

    for p in range(NBUF - 1):
        @pl.when(p < nchunks)
        def _():
            start_chunk(p)

    def per_chunk(m, carry):
        j = carry[0]
        acc0 = carry[1:]
        par = m % NBUF
        b0, base_c = chunk_base(m)
        b1 = jnp.minimum(b0 + CHUNK, r_hi)

        c1, c2 = chunk_copies(m)
        c1.wait()
        c2.wait()

        @pl.when(m + NBUF - 1 < nchunks)
        def _():
            start_chunk(m + NBUF - 1)

        n_started = plsc.all_reduce_population_count(head < b1)
        jB = n_started[0] - 1

        def per_segment(j2, acc):
            bounds = starts_v[pl.ds(j2, NLANE)]
            g0 = jnp.maximum(bounds[0], b0) - base_c
            g1 = jnp.minimum(bounds[1], b1) - base_c

            def row_body(r, acc):
                return tuple(
                    jnp.maximum(acc[u],
                                rows_v[par, r, pl.ds(u * NLANE, NLANE)])
                    for u in range(NVREG))

            UNROLL = 4
            ngrp = (g1 - g0) // UNROLL

            def grp_body(k, acc):
                r = g0 + k * UNROLL
                for i in range(UNROLL):
                    acc = row_body(r + i, acc)
                return acc

            acc = lax.fori_loop(0, ngrp, grp_body, acc)
            acc = lax.fori_loop(g0 + ngrp * UNROLL, g1, row_body, acc)
            for u in range(NVREG):
                out_v[j2, pl.ds(u * NLANE, NLANE)] = acc[u]
            done = bounds[1] <= b1
            return tuple(jnp.where(done, neg, a) for a in acc)

        acc1 = lax.fori_loop(j, jB + 1, per_segment, acc0)
        endB = starts_v[pl.ds(jB, NLANE)]
        j_next = jB + (endB[1] <= b1).astype(jnp.int32)
        return (j_next,) + acc1

    init = (jnp.int32(0),) + tuple(neg for _ in range(NVREG))
    lax.fori_loop(0, nchunks, per_chunk, init)

    pltpu.sync_copy(out_v, out_hbm.at[pl.ds(seg_base, SEG_PER_W)])


def _sc_fused(embs_hbm, idx_hbm, out_hbm, starts_v, out_v, pos_v, probes_v,
              sem, sems):
    def phase_a(idx_v):
        _sc_starts(idx_hbm, idx_v, pos_v, probes_v, starts_v, sem)

    pl.run_scoped(phase_a, pltpu.VMEM((NSLOT * SLOT,), jnp.int32))

    def phase_b(rows_v):
        _sc_segment_max(embs_hbm, out_hbm, starts_v, rows_v, out_v, sems)

    pl.run_scoped(phase_b, pltpu.VMEM((NBUF, CHUNKB, D), jnp.float32))


@jax.jit
def kernel(embs, batch_idx):
    idx = batch_idx.astype(jnp.int32)
    mesh = plsc.VectorSubcoreMesh(core_axis_name="c", subcore_axis_name="s")

    run = pl.kernel(
        _sc_fused,
        mesh=mesh,
        out_type=jax.ShapeDtypeStruct((NSEG, D), jnp.float32),
        scratch_types=[
            pltpu.VMEM((STARTS_PER_W,), jnp.int32),
            pltpu.VMEM((SEG_PER_W, D), jnp.float32),
            pltpu.VMEM((NLANE,), jnp.int32),
            pltpu.VMEM((NLANE,), jnp.int32),
            pltpu.SemaphoreType.DMA,
            pltpu.SemaphoreType.DMA((NBUF,)),
        ],
        compiler_params=pltpu.CompilerParams(needs_layout_passes=False),
    )
    return run(embs, idx)


# speedup vs baseline: 1.0280x; 1.0190x over previous
import functools
import jax
import jax.numpy as jnp
from jax import lax
from jax.experimental import pallas as pl
from jax.experimental.pallas import tpu as pltpu
from jax.experimental.pallas import tpu_sc as plsc

N_ROWS = 100000
D = 128
NSEG = 512
NLANE = 16
NVREG = D // NLANE
NC = 2
NS = 16
NW = NC * NS
SEG_PER_W = NSEG // NW
STARTS_PER_W = 32
CHUNK = 320
CHUNKB = CHUNK + 8
NBUF = 3
NSEARCH = 17
SLOT = 6400
NSLOT = 16
LAST_SLOT = N_ROWS - SLOT * (NSLOT - 1)

NEG_INF = float("-inf")


def _worker_id():
    return lax.axis_index("s") * NC + lax.axis_index("c")


def _sc_starts(idx_hbm, idx_v, pos_v, probes_v, out_v, sem):
    wid = _worker_id()
    lane = lax.broadcasted_iota(jnp.int32, (NLANE,), 0)

    pos_v[pl.ds(0, NLANE)] = jnp.minimum((lane + 1) * SLOT,
                                         SLOT * (NSLOT - 1))
    copy = pltpu.make_async_copy(idx_hbm.at[pos_v], probes_v, sem)
    copy.start()
    copy.wait()
    big = jnp.full((NLANE,), 1 << 30, jnp.int32)
    vals = jnp.where(lane < NSLOT - 1, probes_v[pl.ds(0, NLANE)], big)

    q_min = wid * SEG_PER_W
    q_max = q_min + SEG_PER_W
    k_lo = plsc.all_reduce_population_count(vals < q_min)[0]
    k_hi = plsc.all_reduce_population_count(vals < q_max)[0]
    row_a = k_lo * SLOT
    row_b = jnp.minimum((k_hi + 1) * SLOT, N_ROWS)

    def copy_slot(k2, _):
        dst = pl.multiple_of((k2 - k_lo) * SLOT, 8)
        src = pl.multiple_of(k2 * SLOT, 8)

        @pl.when(k2 < NSLOT - 1)
        def _():
            pltpu.sync_copy(idx_hbm.at[pl.ds(src, SLOT)],
                            idx_v.at[pl.ds(dst, SLOT)])

        @pl.when(k2 >= NSLOT - 1)
        def _():
            pltpu.sync_copy(idx_hbm.at[pl.ds(src, LAST_SLOT)],
                            idx_v.at[pl.ds(dst, LAST_SLOT)])

        return 0

    lax.fori_loop(k_lo, k_hi + 1, copy_slot, 0)

    for half in range(2):
        q = q_min + half * NLANE + lane

        def step(_, carry):
            lo, hi = carry
            mid = jnp.maximum((lo + hi) // 2, row_a)
            probe = plsc.load_gather(idx_v, [mid - row_a])
            lt = probe < q
            return (jnp.where(lt, mid, lo), jnp.where(lt, hi, mid))

        lo0 = jnp.full((NLANE,), -1, jnp.int32) + row_a
        hi0 = jnp.full((NLANE,), 0, jnp.int32) + row_b
        _, hi = lax.fori_loop(0, NSEARCH, step, (lo0, hi0))
        out_v[pl.ds(half * NLANE, NLANE)] = hi


def _sc_segment_max(embs_hbm, out_hbm, starts_v, rows_v, out_v, sems):
    wid = _worker_id()
    seg_base = wid * SEG_PER_W

    neg = jnp.full((NLANE,), NEG_INF, jnp.float32)

    def init_out(j, _):
        for u in range(NVREG):
            out_v[j, pl.ds(u * NLANE, NLANE)] = neg
        return 0

    lax.fori_loop(0, SEG_PER_W, init_out, 0)

    head = starts_v[pl.ds(0, NLANE)]
    tail = starts_v[pl.ds(NLANE, NLANE)]
    r_lo = head[0]
    r_hi = tail[0]
    nchunks = (r_hi - r_lo + CHUNK - 1) // CHUNK

    def chunk_base(k):
        b0 = r_lo + k * CHUNK
        base_c = jnp.minimum((b0 // 8) * 8, N_ROWS - CHUNKB)
        return b0, pl.multiple_of(base_c, 8)

    def chunk_copy(k):
        _, base_c = chunk_base(k)
        buf = k % NBUF
        return pltpu.make_async_copy(embs_hbm.at[pl.ds(base_c, CHUNKB)],
                                     rows_v.at[buf], sems.at[buf])

    def start_chunk(k):
        chunk_copy(k).start()

    for p in range(NBUF - 1):
        @pl.when(p < nchunks)
        def _():
            start_chunk(p)

    def per_chunk(m, carry):
        j = carry[0]
        acc0 = carry[1:]
        par = m % NBUF
        b0, base_c = chunk_base(m)
        b1 = jnp.minimum(b0 + CHUNK, r_hi)

        chunk_copy(m).wait()

        @pl.when(m + NBUF - 1 < nchunks)
        def _():
            start_chunk(m + NBUF - 1)

        n_started = plsc.all_reduce_population_count(head < b1)
        jB = n_started[0] - 1

        def per_segment(j2, acc):
            bounds = starts_v[pl.ds(j2, NLANE)]
            g0 = jnp.maximum(bounds[0], b0) - base_c
            g1 = jnp.minimum(bounds[1], b1) - base_c

            def row_body(r, acc):
                return tuple(
                    jnp.maximum(acc[u],
                                rows_v[par, r, pl.ds(u * NLANE, NLANE)])
                    for u in range(NVREG))

            acc = lax.fori_loop(g0, g1, row_body, acc)
            for u in range(NVREG):
                out_v[j2, pl.ds(u * NLANE, NLANE)] = acc[u]
            done = bounds[1] <= b1
            return tuple(jnp.where(done, neg, a) for a in acc)

        acc1 = lax.fori_loop(j, jB + 1, per_segment, acc0)
        endB = starts_v[pl.ds(jB, NLANE)]
        j_next = jB + (endB[1] <= b1).astype(jnp.int32)
        return (j_next,) + acc1

    init = (jnp.int32(0),) + tuple(neg for _ in range(NVREG))
    lax.fori_loop(0, nchunks, per_chunk, init)

    pltpu.sync_copy(out_v, out_hbm.at[pl.ds(seg_base, SEG_PER_W)])


def _sc_fused(embs_hbm, idx_hbm, out_hbm, starts_v, out_v, pos_v, probes_v,
              sem, sems):
    def phase_a(idx_v):
        _sc_starts(idx_hbm, idx_v, pos_v, probes_v, starts_v, sem)

    pl.run_scoped(phase_a, pltpu.VMEM((NSLOT * SLOT,), jnp.int32))

    def phase_b(rows_v):
        _sc_segment_max(embs_hbm, out_hbm, starts_v, rows_v, out_v, sems)

    pl.run_scoped(phase_b, pltpu.VMEM((NBUF, CHUNKB, D), jnp.float32))


@jax.jit
def kernel(embs, batch_idx):
    idx = batch_idx.astype(jnp.int32)
    mesh = plsc.VectorSubcoreMesh(core_axis_name="c", subcore_axis_name="s")

    run = pl.kernel(
        _sc_fused,
        mesh=mesh,
        out_type=jax.ShapeDtypeStruct((NSEG, D), jnp.float32),
        scratch_types=[
            pltpu.VMEM((STARTS_PER_W,), jnp.int32),
            pltpu.VMEM((SEG_PER_W, D), jnp.float32),
            pltpu.VMEM((NLANE,), jnp.int32),
            pltpu.VMEM((NLANE,), jnp.int32),
            pltpu.SemaphoreType.DMA,
            pltpu.SemaphoreType.DMA((NBUF,)),
        ],
        compiler_params=pltpu.CompilerParams(needs_layout_passes=False),
    )
    return run(embs, idx)
